# baseline (speedup 1.0000x reference)
import jax
import jax.numpy as jnp
from jax.experimental import pallas as pl
from jax.experimental.pallas import tpu as pltpu

EPS = 1e-5
H = W = 32
HP = H + 2
B = 16
N = 16.0 * H * W
BF = jnp.bfloat16


def _stage_slabs(z, slab_ref, c):
    zero_col = jnp.zeros((H, 1, c), BF)
    shifted = (
        jnp.concatenate([zero_col, z[:, : W - 1, :]], axis=1),
        z,
        jnp.concatenate([z[:, 1:, :], zero_col], axis=1),
    )
    zero_row = jnp.zeros((W, 3 * c), BF)
    slab_ref[0, :, 0:3 * c] = zero_row
    for s in range(3):
        slab_ref[1:1 + H, :, s * c:(s + 1) * c] = shifted[s]
    slab_ref[HP - 1, :, 0:3 * c] = zero_row


def _tap_matmuls(slab_ref, w_ref, cin, cout):
    acc = jnp.zeros((H * W, cout), jnp.float32)
    for dy in range(3):
        xs = slab_ref[dy:dy + H, :, 0:3 * cin].reshape(H * W, 3 * cin)
        acc = acc + jnp.dot(xs, w_ref[dy], preferred_element_type=jnp.float32)
    return acc


def _bn_affine(st_ref, g_ref, be_ref):
    mean = st_ref[0] / N
    var = st_ref[1] / N - mean * mean
    scale = g_ref[0] / jnp.sqrt(var + EPS)
    shift = be_ref[0] - mean * scale
    return scale, shift


def _accum_stats(st_ref, y, first):
    s0 = jnp.sum(y, axis=0)
    s1 = jnp.sum(y * y, axis=0)

    @pl.when(first)
    def _():
        st_ref[0] = s0
        st_ref[1] = s1

    @pl.when(jnp.logical_not(first))
    def _():
        st_ref[0] += s0
        st_ref[1] += s1


def _body(x_ref, w1_ref, w2_ref, w3_ref, b1_ref, g1_ref, be1_ref,
          b2_ref, g2_ref, be2_ref, b3_ref, out_ref,
          y1_ref, y2_ref, st1_ref, st2_ref, slab_ref):
    i = pl.program_id(0)
    P = B // 2

    @pl.when(i < P)
    def _conv1():
        for j in range(2):
            _stage_slabs(x_ref[j].astype(BF), slab_ref.at[j], 384)
        for j in range(2):
            y = _tap_matmuls(slab_ref.at[j], w1_ref, 384, 256) + b1_ref[0]
            y1_ref[2 * i + j] = y.astype(BF).reshape(H, W, 256)
            _accum_stats(st1_ref, y, jnp.logical_and(i == 0, j == 0))

    @pl.when(jnp.logical_and(i >= P, i < 2 * P))
    def _conv2():
        scale, shift = _bn_affine(st1_ref, g1_ref, be1_ref)
        for j in range(2):
            b = 2 * (i - P) + j
            z = jnp.maximum(
                y1_ref[b].astype(jnp.float32) * scale + shift, 0.0)
            _stage_slabs(z.astype(BF), slab_ref.at[j], 256)
        for j in range(2):
            b = 2 * (i - P) + j
            y = _tap_matmuls(slab_ref.at[j], w2_ref, 256, 256) + b2_ref[0]
            y2_ref[b] = y.astype(BF).reshape(H, W, 256)
            _accum_stats(st2_ref, y, jnp.logical_and(i == P, j == 0))

    @pl.when(i >= 2 * P)
    def _conv3():
        scale, shift = _bn_affine(st2_ref, g2_ref, be2_ref)
        for j in range(2):
            b = 2 * (i - 2 * P) + j
            z = jnp.maximum(
                y2_ref[b].astype(jnp.float32) * scale + shift, 0.0)
            _stage_slabs(z.astype(BF), slab_ref.at[j], 256)
        for j in range(2):
            y = _tap_matmuls(slab_ref.at[j], w3_ref, 256, 128)[:, 0:1] \
                + b3_ref[0]
            out_ref[j] = jax.nn.sigmoid(y)


@jax.jit
def kernel(dino_features, W1, b1, g1, be1, W2, b2, g2, be2, W3, b3):
    f32 = jnp.float32

    w1r = jnp.transpose(W1.astype(BF), (2, 3, 1, 0)).reshape(3, 3 * 384, 256)
    w2r = jnp.transpose(W2.astype(BF), (2, 3, 1, 0)).reshape(3, 3 * 256, 256)
    w3r = jnp.transpose(W3.astype(BF), (2, 3, 1, 0)).reshape(3, 3 * 256, 1)
    w3r = jnp.pad(w3r, ((0, 0), (0, 0), (0, 127)))

    full = lambda shape: pl.BlockSpec(shape, lambda i: (0,) * len(shape))

    out = pl.pallas_call(
        _body,
        grid=(3 * B // 2,),
        in_specs=[
            pl.BlockSpec((2, H, W, 384),
                         lambda i: (jnp.minimum(i, B // 2 - 1), 0, 0, 0)),
            full((3, 3 * 384, 256)), full((3, 3 * 256, 256)), full((3, 3 * 256, 128)),
            full((1, 256)), full((1, 256)), full((1, 256)),
            full((1, 256)), full((1, 256)), full((1, 256)),
            full((1, 1)),
        ],
        out_specs=pl.BlockSpec((2, H * W, 1),
                               lambda i: (jnp.maximum(i - B, 0), 0, 0)),
        out_shape=jax.ShapeDtypeStruct((B, H * W, 1), f32),
        scratch_shapes=[
            pltpu.VMEM((B, H, W, 256), BF),
            pltpu.VMEM((B, H, W, 256), BF),
            pltpu.VMEM((2, 256), f32),
            pltpu.VMEM((2, 256), f32),
            pltpu.VMEM((2, HP, W, 3 * 384), BF),
        ],
        compiler_params=pltpu.CompilerParams(
            dimension_semantics=("arbitrary",)),
    )(dino_features, w1r, w2r, w3r,
      b1.reshape(1, 256), g1.reshape(1, 256), be1.reshape(1, 256),
      b2.reshape(1, 256), g2.reshape(1, 256), be2.reshape(1, 256),
      b3.reshape(1, 1))

    return out.reshape(B, H, W, 1)


# speedup vs baseline: 1.4673x; 1.4673x over previous
import jax
import jax.numpy as jnp
from jax.experimental import pallas as pl
from jax.experimental.pallas import tpu as pltpu

EPS = 1e-5
H = W = 32
HP = H + 2
B = 16
N = 16.0 * H * W
BF = jnp.bfloat16


def _stage_slabs(z, slab_ref, c):
    zero_col = jnp.zeros((H, 1, c), BF)
    shifted = (
        jnp.concatenate([zero_col, z[:, : W - 1, :]], axis=1),
        z,
        jnp.concatenate([z[:, 1:, :], zero_col], axis=1),
    )
    for s in range(3):
        slab_ref[1:1 + H, :, s * c:(s + 1) * c] = shifted[s]


def _tap_matmuls(slab_ref, w_ref, cin, cout):
    acc = jnp.zeros((H * W, cout), jnp.float32)
    for dy in range(3):
        xs = slab_ref[dy:dy + H, :, 0:3 * cin].reshape(H * W, 3 * cin)
        acc = acc + jnp.dot(xs, w_ref[dy], preferred_element_type=jnp.float32)
    return acc


def _bn_affine(st_ref, g_ref, be_ref):
    mean = st_ref[0] / N
    var = st_ref[1] / N - mean * mean
    scale = g_ref[0] / jnp.sqrt(var + EPS)
    shift = be_ref[0] - mean * scale
    return scale, shift


def _accum_stats(st_ref, y, first):
    s0 = jnp.sum(y, axis=0)
    s1 = jnp.sum(y * y, axis=0)

    @pl.when(first)
    def _():
        st_ref[0] = s0
        st_ref[1] = s1

    @pl.when(jnp.logical_not(first))
    def _():
        st_ref[0] += s0
        st_ref[1] += s1


def _body(x_ref, w1_ref, w2_ref, w3_ref, b1_ref, g1_ref, be1_ref,
          b2_ref, g2_ref, be2_ref, b3_ref, out_ref,
          y1_ref, y2_ref, st1_ref, st2_ref, slab_ref):
    i = pl.program_id(0)

    @pl.when(i == 0)
    def _zero_borders():
        zero_row = jnp.zeros((W, 3 * 384), BF)
        slab_ref[0] = zero_row
        slab_ref[HP - 1] = zero_row

    @pl.when(i < B)
    def _conv1():
        _stage_slabs(x_ref[0].astype(BF), slab_ref, 384)
        y = _tap_matmuls(slab_ref, w1_ref, 384, 256) + b1_ref[0]
        y1_ref[i] = y.astype(BF).reshape(H, W, 256)
        _accum_stats(st1_ref, y, i == 0)

    @pl.when(jnp.logical_and(i >= B, i < 2 * B))
    def _conv2():
        b = i - B
        scale, shift = _bn_affine(st1_ref, g1_ref, be1_ref)
        z = jnp.maximum(
            y1_ref[b] * scale.astype(BF) + shift.astype(BF), BF(0.0))
        _stage_slabs(z, slab_ref, 256)
        y = _tap_matmuls(slab_ref, w2_ref, 256, 256) + b2_ref[0]
        y2_ref[b] = y.astype(BF).reshape(H, W, 256)
        _accum_stats(st2_ref, y, i == B)

    @pl.when(i >= 2 * B)
    def _conv3():
        b = i - 2 * B
        scale, shift = _bn_affine(st2_ref, g2_ref, be2_ref)
        z = jnp.maximum(
            y2_ref[b] * scale.astype(BF) + shift.astype(BF), BF(0.0))
        _stage_slabs(z, slab_ref, 256)
        y = _tap_matmuls(slab_ref, w3_ref, 256, 128)[:, 0:1] + b3_ref[0]
        out_ref[0] = jax.nn.sigmoid(y)


@jax.jit
def kernel(dino_features, W1, b1, g1, be1, W2, b2, g2, be2, W3, b3):
    f32 = jnp.float32

    w1r = jnp.transpose(W1.astype(BF), (2, 3, 1, 0)).reshape(3, 3 * 384, 256)
    w2r = jnp.transpose(W2.astype(BF), (2, 3, 1, 0)).reshape(3, 3 * 256, 256)
    w3r = jnp.transpose(W3.astype(BF), (2, 3, 1, 0)).reshape(3, 3 * 256, 1)
    w3r = jnp.pad(w3r, ((0, 0), (0, 0), (0, 127)))

    full = lambda shape: pl.BlockSpec(shape, lambda i: (0,) * len(shape))

    out = pl.pallas_call(
        _body,
        grid=(3 * B,),
        in_specs=[
            pl.BlockSpec((1, H, W, 384),
                         lambda i: (jnp.minimum(i, B - 1), 0, 0, 0)),
            full((3, 3 * 384, 256)), full((3, 3 * 256, 256)), full((3, 3 * 256, 128)),
            full((1, 256)), full((1, 256)), full((1, 256)),
            full((1, 256)), full((1, 256)), full((1, 256)),
            full((1, 1)),
        ],
        out_specs=pl.BlockSpec((1, H * W, 1),
                               lambda i: (jnp.maximum(i - 2 * B, 0), 0, 0)),
        out_shape=jax.ShapeDtypeStruct((B, H * W, 1), f32),
        scratch_shapes=[
            pltpu.VMEM((B, H, W, 256), BF),
            pltpu.VMEM((B, H, W, 256), BF),
            pltpu.VMEM((2, 256), f32),
            pltpu.VMEM((2, 256), f32),
            pltpu.VMEM((HP, W, 3 * 384), BF),
        ],
        compiler_params=pltpu.CompilerParams(
            dimension_semantics=("arbitrary",)),
    )(dino_features, w1r, w2r, w3r,
      b1.reshape(1, 256), g1.reshape(1, 256), be1.reshape(1, 256),
      b2.reshape(1, 256), g2.reshape(1, 256), be2.reshape(1, 256),
      b3.reshape(1, 1))

    return out.reshape(B, H, W, 1)


# speedup vs baseline: 1.4735x; 1.0042x over previous
import jax
import jax.numpy as jnp
from jax.experimental import pallas as pl
from jax.experimental.pallas import tpu as pltpu

EPS = 1e-5
H = W = 32
HP = H + 2
B = 16
N = 16.0 * H * W
BF = jnp.bfloat16


def _stage_slabs(z, slab_ref, c):
    zero_col = jnp.zeros((H, 1, c), BF)
    shifted = (
        jnp.concatenate([zero_col, z[:, : W - 1, :]], axis=1),
        z,
        jnp.concatenate([z[:, 1:, :], zero_col], axis=1),
    )
    for s in range(3):
        slab_ref[1:1 + H, :, s * c:(s + 1) * c] = shifted[s]


def _tap_matmuls(slab_ref, w_ref, cin, cout):
    halves = []
    for h in range(2):
        r0 = h * (H // 2)
        acc = jnp.zeros((H * W // 2, cout), jnp.float32)
        for dy in range(3):
            xs = slab_ref[r0 + dy:r0 + dy + H // 2, :, 0:3 * cin]
            xs = xs.reshape(H * W // 2, 3 * cin)
            acc = acc + jnp.dot(xs, w_ref[dy],
                                preferred_element_type=jnp.float32)
        halves.append(acc)
    return jnp.concatenate(halves, axis=0)


def _bn_affine(st_ref, g_ref, be_ref):
    mean = st_ref[0] / N
    var = st_ref[1] / N - mean * mean
    scale = g_ref[0] / jnp.sqrt(var + EPS)
    shift = be_ref[0] - mean * scale
    return scale, shift


def _accum_stats(st_ref, y, first):
    s0 = jnp.sum(y, axis=0)
    s1 = jnp.sum(y * y, axis=0)

    @pl.when(first)
    def _():
        st_ref[0] = s0
        st_ref[1] = s1

    @pl.when(jnp.logical_not(first))
    def _():
        st_ref[0] += s0
        st_ref[1] += s1


def _body(x_ref, w1_ref, w2_ref, w3_ref, b1_ref, g1_ref, be1_ref,
          b2_ref, g2_ref, be2_ref, b3_ref, out_ref,
          y1_ref, y2_ref, st1_ref, st2_ref, slab_ref):
    i = pl.program_id(0)

    @pl.when(i == 0)
    def _zero_borders():
        zero_row = jnp.zeros((W, 3 * 384), BF)
        slab_ref[0] = zero_row
        slab_ref[HP - 1] = zero_row

    @pl.when(i < B)
    def _conv1():
        _stage_slabs(x_ref[0].astype(BF), slab_ref, 384)
        y = _tap_matmuls(slab_ref, w1_ref, 384, 256) + b1_ref[0]
        y1_ref[i] = y.astype(BF).reshape(H, W, 256)
        _accum_stats(st1_ref, y, i == 0)

    @pl.when(jnp.logical_and(i >= B, i < 2 * B))
    def _conv2():
        b = i - B
        scale, shift = _bn_affine(st1_ref, g1_ref, be1_ref)
        z = jnp.maximum(
            y1_ref[b] * scale.astype(BF) + shift.astype(BF), BF(0.0))
        _stage_slabs(z, slab_ref, 256)
        y = _tap_matmuls(slab_ref, w2_ref, 256, 256) + b2_ref[0]
        y2_ref[b] = y.astype(BF).reshape(H, W, 256)
        _accum_stats(st2_ref, y, i == B)

    @pl.when(i >= 2 * B)
    def _conv3():
        b = i - 2 * B
        scale, shift = _bn_affine(st2_ref, g2_ref, be2_ref)
        z = jnp.maximum(
            y2_ref[b] * scale.astype(BF) + shift.astype(BF), BF(0.0))
        _stage_slabs(z, slab_ref, 256)
        y = _tap_matmuls(slab_ref, w3_ref, 256, 128)[:, 0:1] + b3_ref[0]
        out_ref[0] = jax.nn.sigmoid(y)


@jax.jit
def kernel(dino_features, W1, b1, g1, be1, W2, b2, g2, be2, W3, b3):
    f32 = jnp.float32

    w1r = jnp.transpose(W1.astype(BF), (2, 3, 1, 0)).reshape(3, 3 * 384, 256)
    w2r = jnp.transpose(W2.astype(BF), (2, 3, 1, 0)).reshape(3, 3 * 256, 256)
    w3r = jnp.transpose(W3.astype(BF), (2, 3, 1, 0)).reshape(3, 3 * 256, 1)
    w3r = jnp.pad(w3r, ((0, 0), (0, 0), (0, 127)))

    full = lambda shape: pl.BlockSpec(shape, lambda i: (0,) * len(shape))

    out = pl.pallas_call(
        _body,
        grid=(3 * B,),
        in_specs=[
            pl.BlockSpec((1, H, W, 384),
                         lambda i: (jnp.minimum(i, B - 1), 0, 0, 0)),
            full((3, 3 * 384, 256)), full((3, 3 * 256, 256)), full((3, 3 * 256, 128)),
            full((1, 256)), full((1, 256)), full((1, 256)),
            full((1, 256)), full((1, 256)), full((1, 256)),
            full((1, 1)),
        ],
        out_specs=pl.BlockSpec((1, H * W, 1),
                               lambda i: (jnp.maximum(i - 2 * B, 0), 0, 0)),
        out_shape=jax.ShapeDtypeStruct((B, H * W, 1), f32),
        scratch_shapes=[
            pltpu.VMEM((B, H, W, 256), BF),
            pltpu.VMEM((B, H, W, 256), BF),
            pltpu.VMEM((2, 256), f32),
            pltpu.VMEM((2, 256), f32),
            pltpu.VMEM((HP, W, 3 * 384), BF),
        ],
        compiler_params=pltpu.CompilerParams(
            dimension_semantics=("arbitrary",)),
    )(dino_features, w1r, w2r, w3r,
      b1.reshape(1, 256), g1.reshape(1, 256), be1.reshape(1, 256),
      b2.reshape(1, 256), g2.reshape(1, 256), be2.reshape(1, 256),
      b3.reshape(1, 1))

    return out.reshape(B, H, W, 1)


# speedup vs baseline: 1.4863x; 1.0087x over previous
import jax
import jax.numpy as jnp
from jax.experimental import pallas as pl
from jax.experimental.pallas import tpu as pltpu

EPS = 1e-5
H = W = 32
HP = H + 2
B = 16
N = 16.0 * H * W
BF = jnp.bfloat16


def _stage_slabs(z, slab_ref, c):
    zero_col = jnp.zeros((H, 1, c), BF)
    shifted = (
        jnp.concatenate([zero_col, z[:, : W - 1, :]], axis=1),
        z,
        jnp.concatenate([z[:, 1:, :], zero_col], axis=1),
    )
    for s in range(3):
        slab_ref[1:1 + H, :, s * c:(s + 1) * c] = shifted[s]


def _tap_matmuls(slab_ref, w_ref, cin, cout):
    halves = []
    for h in range(2):
        r0 = h * (H // 2)
        acc = jnp.zeros((H * W // 2, cout), jnp.float32)
        for dy in range(3):
            xs = slab_ref[r0 + dy:r0 + dy + H // 2, :, 0:3 * cin]
            xs = xs.reshape(H * W // 2, 3 * cin)
            acc = acc + jnp.dot(xs, w_ref[dy],
                                preferred_element_type=jnp.float32)
        halves.append(acc)
    return halves


def _bn_affine(st_ref, g_ref, be_ref):
    mean = st_ref[0] / N
    var = st_ref[1] / N - mean * mean
    scale = g_ref[0] / jnp.sqrt(var + EPS)
    shift = be_ref[0] - mean * scale
    return scale, shift


def _accum_stats(st_ref, s0, s1, first):
    @pl.when(first)
    def _():
        st_ref[0] = s0
        st_ref[1] = s1

    @pl.when(jnp.logical_not(first))
    def _():
        st_ref[0] += s0
        st_ref[1] += s1


def _body(x_ref, w1_ref, w2_ref, w3_ref, b1_ref, g1_ref, be1_ref,
          b2_ref, g2_ref, be2_ref, b3_ref, out_ref,
          y1_ref, y2_ref, st1_ref, st2_ref, slab_ref):
    i = pl.program_id(0)

    @pl.when(i == 0)
    def _zero_borders():
        zero_row = jnp.zeros((W, 3 * 384), BF)
        slab_ref[0] = zero_row
        slab_ref[HP - 1] = zero_row

    @pl.when(i < B)
    def _conv1():
        _stage_slabs(x_ref[0].astype(BF), slab_ref, 384)
        s0, s1 = 0.0, 0.0
        for h, acc in enumerate(_tap_matmuls(slab_ref, w1_ref, 384, 256)):
            y = acc + b1_ref[0]
            y1_ref[i, h * (H // 2):(h + 1) * (H // 2)] = \
                y.astype(BF).reshape(H // 2, W, 256)
            s0 = s0 + jnp.sum(y, axis=0)
            s1 = s1 + jnp.sum(y * y, axis=0)
        _accum_stats(st1_ref, s0, s1, i == 0)

    @pl.when(jnp.logical_and(i >= B, i < 2 * B))
    def _conv2():
        b = i - B
        scale, shift = _bn_affine(st1_ref, g1_ref, be1_ref)
        z = jnp.maximum(
            y1_ref[b] * scale.astype(BF) + shift.astype(BF), BF(0.0))
        _stage_slabs(z, slab_ref, 256)
        s0, s1 = 0.0, 0.0
        for h, acc in enumerate(_tap_matmuls(slab_ref, w2_ref, 256, 256)):
            y = acc + b2_ref[0]
            y2_ref[b, h * (H // 2):(h + 1) * (H // 2)] = \
                y.astype(BF).reshape(H // 2, W, 256)
            s0 = s0 + jnp.sum(y, axis=0)
            s1 = s1 + jnp.sum(y * y, axis=0)
        _accum_stats(st2_ref, s0, s1, i == B)

    @pl.when(i >= 2 * B)
    def _conv3():
        b = i - 2 * B
        scale, shift = _bn_affine(st2_ref, g2_ref, be2_ref)
        z = jnp.maximum(
            y2_ref[b] * scale.astype(BF) + shift.astype(BF), BF(0.0))
        _stage_slabs(z, slab_ref, 256)
        for h, acc in enumerate(_tap_matmuls(slab_ref, w3_ref, 256, 128)):
            y = acc[:, 0:1] + b3_ref[0]
            out_ref[0, h * (H * W // 2):(h + 1) * (H * W // 2)] = \
                jax.nn.sigmoid(y)


@jax.jit
def kernel(dino_features, W1, b1, g1, be1, W2, b2, g2, be2, W3, b3):
    f32 = jnp.float32

    w1r = jnp.transpose(W1.astype(BF), (2, 3, 1, 0)).reshape(3, 3 * 384, 256)
    w2r = jnp.transpose(W2.astype(BF), (2, 3, 1, 0)).reshape(3, 3 * 256, 256)
    w3r = jnp.transpose(W3.astype(BF), (2, 3, 1, 0)).reshape(3, 3 * 256, 1)
    w3r = jnp.pad(w3r, ((0, 0), (0, 0), (0, 127)))

    full = lambda shape: pl.BlockSpec(shape, lambda i: (0,) * len(shape))

    out = pl.pallas_call(
        _body,
        grid=(3 * B,),
        in_specs=[
            pl.BlockSpec((1, H, W, 384),
                         lambda i: (jnp.minimum(i, B - 1), 0, 0, 0)),
            full((3, 3 * 384, 256)), full((3, 3 * 256, 256)), full((3, 3 * 256, 128)),
            full((1, 256)), full((1, 256)), full((1, 256)),
            full((1, 256)), full((1, 256)), full((1, 256)),
            full((1, 1)),
        ],
        out_specs=pl.BlockSpec((1, H * W, 1),
                               lambda i: (jnp.maximum(i - 2 * B, 0), 0, 0)),
        out_shape=jax.ShapeDtypeStruct((B, H * W, 1), f32),
        scratch_shapes=[
            pltpu.VMEM((B, H, W, 256), BF),
            pltpu.VMEM((B, H, W, 256), BF),
            pltpu.VMEM((2, 256), f32),
            pltpu.VMEM((2, 256), f32),
            pltpu.VMEM((HP, W, 3 * 384), BF),
        ],
        compiler_params=pltpu.CompilerParams(
            dimension_semantics=("arbitrary",)),
    )(dino_features, w1r, w2r, w3r,
      b1.reshape(1, 256), g1.reshape(1, 256), be1.reshape(1, 256),
      b2.reshape(1, 256), g2.reshape(1, 256), be2.reshape(1, 256),
      b3.reshape(1, 1))

    return out.reshape(B, H, W, 1)
